# baseline (speedup 1.0000x reference)
import functools

import jax
import jax.numpy as jnp
from jax import lax
from jax.experimental import pallas as pl
from jax.experimental.pallas import tpu as pltpu
from jax.experimental.pallas import tpu_sc as plsc

TAU = 0.07
B, C, H, W = 8, 96, 128, 128
K = 21
KP = 32
HW = H * W
MTOT = B * HW
NC, NS = 2, 16
NW = NC * NS
CAP = 512
SLICE = MTOT // NW
NVEC = SLICE // 16
CHUNK = 128
NCH = CAP // CHUNK
IDXN = C * CHUNK
NBLK = NW * NCH


BLKP = 2048


def _bitpack_body(gt_ref, out_ref):
    g = gt_ref[...]
    kio = lax.broadcasted_iota(jnp.int32, (B, K, BLKP), 1)
    bit = jnp.where(g == 1, jnp.left_shift(jnp.int32(1), kio), 0)
    out_ref[...] = jnp.sum(bit, axis=1)


def _sc_body(mpack_ref, feat_ref, g_ref, bits_ref,
             mbuf, pixbuf, bitsbuf, wbuf, idxbuf, gchunk, dsem):
    cid = lax.axis_index("c")
    sid = lax.axis_index("s")
    wid = sid * NC + cid
    base = wid * SLICE
    pltpu.sync_copy(mpack_ref.at[pl.ds(base, SLICE)], mbuf)

    zero16 = jnp.zeros((16,), jnp.int32)

    def zbody(j, _):
        pixbuf[pl.ds(j * 16, 16)] = zero16
        bitsbuf[pl.ds(j * 16, 16)] = zero16
        return 0

    lax.fori_loop(0, CAP // 16, zbody, 0)

    lane = lax.broadcasted_iota(jnp.int32, (16,), 0)

    def scan_body(i, off):
        v = mbuf[pl.ds(i * 16, 16)]
        msk = v != 0
        mi = jnp.where(msk, 1, 0)
        cs = plsc.cumsum(mi)
        idx = off + cs - 1
        safe = jnp.logical_and(msk, idx < CAP)
        pix = base + i * 16 + lane
        plsc.store_scatter(pixbuf, [idx], pix, mask=safe)
        plsc.store_scatter(bitsbuf, [idx], v, mask=safe)
        return off + jnp.sum(mi)

    cnt = lax.fori_loop(0, NVEC, scan_body, jnp.int32(0))
    cnt = jnp.minimum(cnt, CAP)

    pltpu.sync_copy(bitsbuf, bits_ref.at[pl.ds(wid * CAP, CAP)])

    def wb_body(j, _):
        pix = pixbuf[pl.ds(j * 16, 16)]
        wbuf[pl.ds(j * 16, 16)] = (pix >> 14) * (C * HW) + (pix & (HW - 1))
        return 0

    lax.fori_loop(0, CAP // 16, wb_body, 0)

    for q in range(NCH):
        @pl.when(cnt > q * CHUNK)
        def _():
            def idx_body(c, _):
                col = c * HW
                for j in range(CHUNK // 16):
                    wv = wbuf[pl.ds(q * CHUNK + j * 16, 16)]
                    idxbuf[pl.ds(c * CHUNK + j * 16, 16)] = wv + col
                return 0

            lax.fori_loop(0, C, idx_body, 0)

            def fire(c, _):
                sl = pl.ds(c * CHUNK, CHUNK)
                pltpu.async_copy(feat_ref.at[idxbuf.at[sl]],
                                 gchunk.at[sl], dsem)
                return 0

            lax.fori_loop(0, C, fire, 0)
            pltpu.make_async_copy(feat_ref.at[pl.ds(0, IDXN)],
                                  gchunk, dsem).wait()
            pltpu.sync_copy(gchunk, g_ref.at[wid * NCH + q])


def _loss_body(bits_ref, g_ref, out_ref, k0_ref, acc_ref):
    p = pl.program_id(0)
    blk = pl.program_id(1)

    @pl.when(jnp.logical_and(p == 0, blk == 0))
    def _():
        k0_ref[...] = jnp.zeros_like(k0_ref)
        acc_ref[0] = 0.0
        acc_ref[1] = 0.0

    bits = bits_ref[0, 0]
    g = g_ref[0]
    valid = (bits != 0).reshape(1, CHUNK)
    gm = jnp.where(valid, g, 0.0)
    kio = lax.broadcasted_iota(jnp.int32, (KP, CHUNK), 0)
    wsel = ((jnp.broadcast_to(bits.reshape(1, CHUNK), (KP, CHUNK))
             >> kio) & 1).astype(jnp.float32)

    @pl.when(p == 0)
    def _():
        k0_ref[...] += lax.dot_general(
            gm, wsel, (((1,), (1,)), ((), ())),
            preferred_element_type=jnp.float32)
        acc_ref[1] += jnp.sum(wsel)

    @pl.when(p == 1)
    def _():
        k0 = k0_ref[...]
        k0n = k0 / jnp.maximum(
            jnp.sqrt(jnp.sum(k0 * k0, axis=0, keepdims=True)), 1e-12)
        s = lax.dot_general(k0n, gm, (((0,), (0,)), ((), ())),
                            preferred_element_type=jnp.float32)
        invf = 1.0 / jnp.maximum(
            jnp.sqrt(jnp.sum(gm * gm, axis=0, keepdims=True)), 1e-12)
        st = s * invf / TAU
        kmask = kio < K
        e = jnp.where(kmask, jnp.exp(st), 0.0)
        denom = jnp.sum(e, axis=0, keepdims=True)
        acc_ref[0] += jnp.sum(wsel * (st - jnp.log(denom)))

    @pl.when(jnp.logical_and(p == 1, blk == NBLK - 1))
    def _():
        out_ref[...] = -(acc_ref[0] / acc_ref[1]) * jnp.ones((1, 1),
                                                             jnp.float32)


_sc_call = pl.kernel(
    _sc_body,
    out_type=[
        jax.ShapeDtypeStruct((NBLK, IDXN), jnp.float32),
        jax.ShapeDtypeStruct((NW * CAP,), jnp.int32),
    ],
    mesh=plsc.VectorSubcoreMesh(core_axis_name="c", subcore_axis_name="s",
                                num_cores=NC, num_subcores=NS),
    compiler_params=pltpu.CompilerParams(needs_layout_passes=False),
    scratch_types=[
        pltpu.VMEM((SLICE,), jnp.int32),
        pltpu.VMEM((CAP,), jnp.int32),
        pltpu.VMEM((CAP,), jnp.int32),
        pltpu.VMEM((CAP,), jnp.int32),
        pltpu.VMEM((IDXN,), jnp.int32),
        pltpu.VMEM((IDXN,), jnp.float32),
        pltpu.SemaphoreType.DMA,
    ],
)


@jax.jit
def kernel(feat, gt):
    gtr = gt.reshape(B, K, HW)
    mpack = pl.pallas_call(
        _bitpack_body,
        grid=(HW // BLKP,),
        in_specs=[pl.BlockSpec((B, K, BLKP), lambda j: (0, 0, j))],
        out_specs=pl.BlockSpec((B, BLKP), lambda j: (0, j)),
        out_shape=jax.ShapeDtypeStruct((B, HW), jnp.int32),
    )(gtr)

    g, bits = _sc_call(mpack.reshape(MTOT), feat.reshape(B * C * HW))

    gb = g.reshape(NBLK, C, CHUNK)
    bitsb = bits.reshape(NBLK, 1, CHUNK)
    loss = pl.pallas_call(
        _loss_body,
        grid=(2, NBLK),
        in_specs=[
            pl.BlockSpec((1, 1, CHUNK), lambda p, b: (b, 0, 0)),
            pl.BlockSpec((1, C, CHUNK), lambda p, b: (b, 0, 0)),
        ],
        out_specs=pl.BlockSpec((1, 1), lambda p, b: (0, 0)),
        out_shape=jax.ShapeDtypeStruct((1, 1), jnp.float32),
        scratch_shapes=[
            pltpu.VMEM((C, KP), jnp.float32),
            pltpu.SMEM((2,), jnp.float32),
        ],
    )(bitsb, gb)
    return loss.reshape(1)


# speedup vs baseline: 2.5120x; 2.5120x over previous
import functools

import jax
import jax.numpy as jnp
from jax import lax
from jax.experimental import pallas as pl
from jax.experimental.pallas import tpu as pltpu
from jax.experimental.pallas import tpu_sc as plsc

TAU = 0.07
B, C, H, W = 8, 96, 128, 128
K = 21
KP = 32
HW = H * W
MTOT = B * HW
NC, NS = 2, 16
NW = NC * NS
CAP = 512
SLICE = MTOT // NW
NVEC = SLICE // 16
CHUNK = 128
NCH = CAP // CHUNK
IDXN = C * CHUNK
TOT = NW * CAP
BLKC = 2048


BLKP = 2048


def _bitpack_body(gt_ref, out_ref):
    g = gt_ref[...]
    kio = lax.broadcasted_iota(jnp.int32, (B, K, BLKP), 1)
    bit = jnp.where(g == 1, jnp.left_shift(jnp.int32(1), kio), 0)
    out_ref[...] = jnp.sum(bit, axis=1)


def _sc_body(mpack_ref, feat_ref, g_ref, bits_ref,
             mbuf, pixbuf, bitsbuf, wbuf, idxbuf, gchunk, dsem):
    cid = lax.axis_index("c")
    sid = lax.axis_index("s")
    wid = sid * NC + cid
    base = wid * SLICE
    pltpu.sync_copy(mpack_ref.at[pl.ds(base, SLICE)], mbuf)

    zero16 = jnp.zeros((16,), jnp.int32)

    def zbody(j, _):
        pixbuf[pl.ds(j * 16, 16)] = zero16
        bitsbuf[pl.ds(j * 16, 16)] = zero16
        return 0

    lax.fori_loop(0, CAP // 16, zbody, 0)

    lane = lax.broadcasted_iota(jnp.int32, (16,), 0)

    def scan_body(i, off):
        v = mbuf[pl.ds(i * 16, 16)]
        msk = v != 0
        mi = jnp.where(msk, 1, 0)
        cs = plsc.cumsum(mi)
        idx = off + cs - 1
        safe = jnp.logical_and(msk, idx < CAP)
        pix = base + i * 16 + lane
        plsc.store_scatter(pixbuf, [idx], pix, mask=safe)
        plsc.store_scatter(bitsbuf, [idx], v, mask=safe)
        return off + jnp.sum(mi)

    cnt = lax.fori_loop(0, NVEC, scan_body, jnp.int32(0))
    cnt = jnp.minimum(cnt, CAP)

    pltpu.sync_copy(bitsbuf, bits_ref.at[pl.ds(wid * CAP, CAP)])

    def wb_body(j, _):
        pix = pixbuf[pl.ds(j * 16, 16)]
        wbuf[pl.ds(j * 16, 16)] = (pix >> 14) * (C * HW) + (pix & (HW - 1))
        return 0

    lax.fori_loop(0, CAP // 16, wb_body, 0)

    for q in range(NCH):
        @pl.when(cnt > q * CHUNK)
        def _():
            def idx_body(c, _):
                col = c * HW
                for j in range(CHUNK // 16):
                    wv = wbuf[pl.ds(q * CHUNK + j * 16, 16)]
                    idxbuf[pl.ds(c * CHUNK + j * 16, 16)] = wv + col
                return 0

            lax.fori_loop(0, C, idx_body, 0)

            def fire(c, _):
                sl = pl.ds(c * CHUNK, CHUNK)
                pltpu.async_copy(feat_ref.at[idxbuf.at[sl]],
                                 gchunk.at[c], dsem)
                return 0

            lax.fori_loop(0, C, fire, 0)
            pltpu.make_async_copy(feat_ref.at[pl.ds(0, IDXN)],
                                  idxbuf, dsem).wait()
            pltpu.sync_copy(
                gchunk,
                g_ref.at[:, pl.ds(wid * CAP + q * CHUNK, CHUNK)])


def _loss_body(bits_ref, g_ref, out_ref, k0_ref, acc_ref):
    p = pl.program_id(0)
    blk = pl.program_id(1)

    @pl.when(jnp.logical_and(p == 0, blk == 0))
    def _():
        k0_ref[...] = jnp.zeros_like(k0_ref)
        acc_ref[0] = 0.0
        acc_ref[1] = 0.0

    bits = bits_ref[0]
    g = g_ref[...]
    valid = (bits != 0).reshape(1, BLKC)
    gm = jnp.where(valid, g, 0.0)
    kio = lax.broadcasted_iota(jnp.int32, (KP, BLKC), 0)
    wsel = ((jnp.broadcast_to(bits.reshape(1, BLKC), (KP, BLKC))
             >> kio) & 1).astype(jnp.float32)

    @pl.when(p == 0)
    def _():
        k0_ref[...] += lax.dot_general(
            gm, wsel, (((1,), (1,)), ((), ())),
            preferred_element_type=jnp.float32)
        acc_ref[1] += jnp.sum(wsel)

    @pl.when(p == 1)
    def _():
        k0 = k0_ref[...]
        k0n = k0 / jnp.maximum(
            jnp.sqrt(jnp.sum(k0 * k0, axis=0, keepdims=True)), 1e-12)
        s = lax.dot_general(k0n, gm, (((0,), (0,)), ((), ())),
                            preferred_element_type=jnp.float32)
        invf = 1.0 / jnp.maximum(
            jnp.sqrt(jnp.sum(gm * gm, axis=0, keepdims=True)), 1e-12)
        st = s * invf / TAU
        kmask = kio < K
        e = jnp.where(kmask, jnp.exp(st), 0.0)
        denom = jnp.sum(e, axis=0, keepdims=True)
        acc_ref[0] += jnp.sum(wsel * (st - jnp.log(denom)))

    @pl.when(jnp.logical_and(p == 1, blk == TOT // BLKC - 1))
    def _():
        out_ref[...] = -(acc_ref[0] / acc_ref[1]) * jnp.ones((1, 1),
                                                             jnp.float32)


_sc_call = pl.kernel(
    _sc_body,
    out_type=[
        jax.ShapeDtypeStruct((C, TOT), jnp.float32),
        jax.ShapeDtypeStruct((TOT,), jnp.int32),
    ],
    mesh=plsc.VectorSubcoreMesh(core_axis_name="c", subcore_axis_name="s",
                                num_cores=NC, num_subcores=NS),
    compiler_params=pltpu.CompilerParams(needs_layout_passes=False),
    scratch_types=[
        pltpu.VMEM((SLICE,), jnp.int32),
        pltpu.VMEM((CAP,), jnp.int32),
        pltpu.VMEM((CAP,), jnp.int32),
        pltpu.VMEM((CAP,), jnp.int32),
        pltpu.VMEM((IDXN,), jnp.int32),
        pltpu.VMEM((C, CHUNK), jnp.float32),
        pltpu.SemaphoreType.DMA,
    ],
)


@jax.jit
def kernel(feat, gt):
    gtr = gt.reshape(B, K, HW)
    mpack = pl.pallas_call(
        _bitpack_body,
        grid=(HW // BLKP,),
        in_specs=[pl.BlockSpec((B, K, BLKP), lambda j: (0, 0, j))],
        out_specs=pl.BlockSpec((B, BLKP), lambda j: (0, j)),
        out_shape=jax.ShapeDtypeStruct((B, HW), jnp.int32),
    )(gtr)

    g, bits = _sc_call(mpack.reshape(MTOT), feat.reshape(B * C * HW))

    bitsb = bits.reshape(1, TOT)
    loss = pl.pallas_call(
        _loss_body,
        grid=(2, TOT // BLKC),
        in_specs=[
            pl.BlockSpec((1, BLKC), lambda p, b: (0, b)),
            pl.BlockSpec((C, BLKC), lambda p, b: (0, b)),
        ],
        out_specs=pl.BlockSpec((1, 1), lambda p, b: (0, 0)),
        out_shape=jax.ShapeDtypeStruct((1, 1), jnp.float32),
        scratch_shapes=[
            pltpu.VMEM((C, KP), jnp.float32),
            pltpu.SMEM((2,), jnp.float32),
        ],
    )(bitsb, g)
    return loss.reshape(1)
